# Initial kernel scaffold; baseline (speedup 1.0000x reference)
import jax
import jax.numpy as jnp
from jax.experimental import pallas as pl


def kernel(inputs, codebook):
    raise NotImplementedError("write your pallas kernel here")


# speedup vs baseline: 6.2609x; 6.2609x over previous
import jax
import jax.numpy as jnp
from jax import lax
from jax.experimental import pallas as pl
from jax.experimental.pallas import tpu as pltpu

_VOCAB = 8192
_D = 32
_N = 4096
_BETA = 0.25

_ABLK = 512
_NA = _VOCAB // _ABLK
_OBLK = 1024
_NO = _VOCAB // _OBLK


def _argmin_body(x_ref, c_ref, idx_ref, minv_ref):
    j = pl.program_id(0)
    x = x_ref[...]
    c = c_ref[...]
    rs = jnp.sum(x * x, axis=1, keepdims=True)
    csq = c * c
    ones = jnp.ones((1, _D), jnp.float32)
    cs = lax.dot_general(ones, csq, (((1,), (1,)), ((), ())),
                         precision=lax.Precision.HIGHEST)
    mm = lax.dot_general(x, c, (((1,), (1,)), ((), ())))
    d = (rs + cs) - 2.0 * mm
    bmin = jnp.min(d, axis=1, keepdims=True)
    col = lax.broadcasted_iota(jnp.int32, (_N, _ABLK), 1)
    cand = jnp.where(d == bmin, col, jnp.int32(2 ** 30))
    bidx = jnp.min(cand, axis=1, keepdims=True) + j * _ABLK

    @pl.when(j == 0)
    def _():
        minv_ref[...] = bmin
        idx_ref[...] = bidx

    @pl.when(j > 0)
    def _():
        upd = bmin < minv_ref[...]
        minv_ref[...] = jnp.where(upd, bmin, minv_ref[...])
        idx_ref[...] = jnp.where(upd, bidx, idx_ref[...])


def _emit_body(idx_ref, x_ref, c_ref, oh_ref, q_ref, loss_ref, ppl_ref,
               cnt_ref):
    j = pl.program_id(0)
    idx = idx_ref[...]
    col = lax.broadcasted_iota(jnp.int32, (_N, _OBLK), 1) + j * _OBLK
    oh = jnp.where(idx == col, 1.0, 0.0).astype(jnp.float32)
    oh_ref[...] = oh
    cnt_ref[:, pl.ds(j * _OBLK, _OBLK)] = jnp.sum(oh, axis=0, keepdims=True)
    qpart = lax.dot_general(oh, c_ref[...], (((1,), (0,)), ((), ())))

    @pl.when(j == 0)
    def _():
        q_ref[...] = qpart

    @pl.when(j > 0)
    def _():
        q_ref[...] = q_ref[...] + qpart

    @pl.when(j == _NO - 1)
    def _():
        x = x_ref[...]
        q = q_ref[...]
        diff = q - x
        s = jnp.sum(jnp.sum(diff * diff, axis=1, keepdims=True), axis=0,
                    keepdims=True)
        m = s * (1.0 / (_N * _D))
        loss_ref[...] = m + _BETA * m
        avg = cnt_ref[...] * (1.0 / _N)
        ent = jnp.sum(avg * jnp.log(avg + 1e-10), axis=1, keepdims=True)
        ppl_ref[...] = jnp.exp(-ent)
        q_ref[...] = x + (q - x)


def _argmin_call(xf, codebook):
    return pl.pallas_call(
        _argmin_body,
        grid=(_NA,),
        in_specs=[
            pl.BlockSpec((_N, _D), lambda j: (0, 0)),
            pl.BlockSpec((_ABLK, _D), lambda j: (j, 0)),
        ],
        out_specs=pl.BlockSpec((_N, 1), lambda j: (0, 0)),
        out_shape=jax.ShapeDtypeStruct((_N, 1), jnp.int32),
        scratch_shapes=[pltpu.VMEM((_N, 1), jnp.float32)],
    )(xf, codebook)


def _emit_call(idx2, xf, codebook):
    return pl.pallas_call(
        _emit_body,
        grid=(_NO,),
        in_specs=[
            pl.BlockSpec((_N, 1), lambda j: (0, 0)),
            pl.BlockSpec((_N, _D), lambda j: (0, 0)),
            pl.BlockSpec((_OBLK, _D), lambda j: (j, 0)),
        ],
        out_specs=[
            pl.BlockSpec((_N, _OBLK), lambda j: (0, j)),
            pl.BlockSpec((_N, _D), lambda j: (0, 0)),
            pl.BlockSpec((1, 1), lambda j: (0, 0)),
            pl.BlockSpec((1, 1), lambda j: (0, 0)),
        ],
        out_shape=[
            jax.ShapeDtypeStruct((_N, _VOCAB), jnp.float32),
            jax.ShapeDtypeStruct((_N, _D), jnp.float32),
            jax.ShapeDtypeStruct((1, 1), jnp.float32),
            jax.ShapeDtypeStruct((1, 1), jnp.float32),
        ],
        scratch_shapes=[pltpu.VMEM((1, _VOCAB), jnp.float32)],
    )(idx2, xf, codebook)


def kernel(inputs, codebook):
    x4 = jnp.transpose(inputs, (0, 2, 3, 1))
    xf = x4.reshape(_N, _D)
    idx2 = _argmin_call(xf, codebook)
    oh, q, loss11, ppl11 = _emit_call(idx2, xf, codebook)
    quantized_out = jnp.transpose(q.reshape(x4.shape), (0, 3, 1, 2))
    return (loss11[0, 0], quantized_out, ppl11[0, 0], oh)
